# baseline (speedup 1.0000x reference)
import functools

import jax
import jax.numpy as jnp
from jax import lax
from jax.experimental import pallas as pl
from jax.experimental.pallas import tpu as pltpu
from jax.experimental.pallas import tpu_sc as plsc

B, N, D = 1024, 200, 128
_SCALE = float(D) ** 0.5

_NW = 32
_BPW = B // _NW
_CK = 40
_NCK = N // _CK
_T = (_BPW // 2) * _NCK
_RW = 224


def _sc_body(len_hbm, ids_hbm, r_hbm, pe_hbm, posm_hbm, t7_hbm,
             l2_hbm, ue_hbm, me_hbm, mo_hbm,
             posm_v, t7_v, idsb_v, rb_v, pe_v, out_v, me_v, mo_v, rch_v,
             len_v, in_sem, out_sem):
    c = lax.axis_index("c")
    s = lax.axis_index("s")
    w = s * 2 + c
    b0 = pl.multiple_of(w * _BPW, 8)

    pltpu.sync_copy(posm_hbm, posm_v)
    pltpu.sync_copy(t7_hbm, t7_v)

    pltpu.sync_copy(len_hbm.at[pl.ds(b0, _BPW)], len_v)
    for g in range(_BPW // 16):
        len_v[pl.ds(g * 16, 16)] = len_v[pl.ds(g * 16, 16)] * 2
    pltpu.sync_copy(len_v, l2_hbm.at[pl.ds(b0, _BPW)])

    def in_copies(t, slot):
        bp = t // _NCK
        i0 = pl.multiple_of((t % _NCK) * _CK, 8)
        return tuple(
            pltpu.make_async_copy(pe_hbm.at[b0 + 2 * bp + q, pl.ds(i0, _CK)],
                                  pe_v.at[pl.ds(pl.multiple_of(
                                      (slot * 2 + q) * _CK, 8), _CK)],
                                  in_sem.at[slot])
            for q in (0, 1))

    def out_copies(t, slot):
        bp = t // _NCK
        i0 = pl.multiple_of((t % _NCK) * _CK, 8)
        sb = pl.multiple_of(lax.rem(bp, 2) * 2 * _RW, 8)
        dmas = []
        for q in (0, 1):
            b = b0 + 2 * bp + q
            fo = pl.multiple_of(b * N + i0, 8)
            mb = pl.multiple_of(sb + q * _RW + i0, 8)
            dmas.append(pltpu.make_async_copy(
                out_v.at[pl.ds(pl.multiple_of(
                    (slot * 2 + q) * 2 * _CK, 8), 2 * _CK)],
                ue_hbm.at[b, pl.ds(2 * i0, 2 * _CK)],
                out_sem.at[slot]))
            dmas.append(pltpu.make_async_copy(me_v.at[pl.ds(mb, _CK)],
                                              me_hbm.at[pl.ds(fo, _CK)],
                                              out_sem.at[slot]))
            dmas.append(pltpu.make_async_copy(mo_v.at[pl.ds(mb, _CK)],
                                              mo_hbm.at[pl.ds(fo, _CK)],
                                              out_sem.at[slot]))
        return tuple(dmas)

    def compute(t, slot):
        bp = t // _NCK
        i0 = (t % _NCK) * _CK
        sb = lax.rem(bp, 2) * 2 * _RW

        @pl.when((i0 == 0) & (lax.rem(bp, 4) == 0))
        def _stage_window():
            wb = pl.multiple_of(b0 + 2 * bp, 8)
            pltpu.sync_copy(ids_hbm.at[pl.ds(wb, 8)], idsb_v)
            pltpu.sync_copy(r_hbm.at[pl.ds(wb, 8)], rb_v)

        @pl.when(i0 == 0)
        def _stage_row():
            for q in (0, 1):
                sq = sb + q * _RW
                wrow = lax.rem(2 * bp, 8) + q
                for goff in list(range(0, N - 16, 16)) + [N - 16]:
                    ids16 = idsb_v[wrow, pl.ds(goff, 16)]
                    r16 = rb_v[wrow, pl.ds(goff, 16)]
                    me_v[pl.ds(sq + goff, 16)] = jnp.where(
                        ids16 != 0, 1.0, 0.0).astype(jnp.float32)
                    mo_v[pl.ds(sq + goff, 16)] = jnp.where(
                        (r16 != 0) & (r16 != 6), 1.0,
                        0.0).astype(jnp.float32)
                    rch_v[pl.ds(sq + goff, 16)] = r16

        @plsc.parallel_loop(0, _CK, 1, unroll=2)
        def pair(j):
            pme = [posm_v[i0 + j, pl.ds(k * 16, 16)] for k in range(8)]
            pmo = [posm_v[i0 + j, pl.ds(D + k * 16, 16)] for k in range(8)]
            for q in (0, 1):
                moff = sb + q * _RW + i0 + j
                me_s = lax.broadcast(me_v[pl.ds(moff, 16)][0], (16,))
                mo_s = lax.broadcast(mo_v[pl.ds(moff, 16)][0], (16,))
                r = rch_v[pl.ds(moff, 16)][0]
                rb = (slot * 2 + q) * _CK
                rb2 = (slot * 2 + q) * 2 * _CK
                for k in range(8):
                    ev = pe_v[rb + j, pl.ds(k * 16, 16)] * _SCALE
                    ev = (ev + pme[k]) * me_s
                    out_v[rb2 + 2 * j, pl.ds(k * 16, 16)] = ev
                    ov = (t7_v[r, pl.ds(k * 16, 16)] + pmo[k]) * mo_s
                    out_v[rb2 + 2 * j + 1, pl.ds(k * 16, 16)] = ov

    for dma in in_copies(0, 0):
        dma.start()

    def step(t, _):
        slot = lax.rem(t, 2)

        @pl.when(t + 1 < _T)
        def _prefetch():
            for dma in in_copies(t + 1, 1 - slot):
                dma.start()

        for dma in in_copies(t, slot):
            dma.wait()

        @pl.when(t >= 2)
        def _drain_out():
            for dma in out_copies(t - 2, slot):
                dma.wait()

        compute(t, slot)
        for dma in out_copies(t, slot):
            dma.start()
        return _

    lax.fori_loop(0, _T, step, 0)
    for dma in out_copies(_T - 2, 0):
        dma.wait()
    for dma in out_copies(_T - 1, 1):
        dma.wait()


@jax.jit
def _run_sc(past_lengths, past_ids, ratings, past_embeddings, posm, t7):
    mesh = plsc.VectorSubcoreMesh(core_axis_name="c", subcore_axis_name="s")
    f = functools.partial(
        pl.kernel,
        mesh=mesh,
        out_type=[
            jax.ShapeDtypeStruct((B,), jnp.int32),
            jax.ShapeDtypeStruct((B, 2 * N, D), jnp.float32),
            jax.ShapeDtypeStruct((B * N,), jnp.float32),
            jax.ShapeDtypeStruct((B * N,), jnp.float32),
        ],
        scratch_types=[
            pltpu.VMEM((N, 2 * D), jnp.float32),
            pltpu.VMEM((8, D), jnp.float32),
            pltpu.VMEM((8, N), jnp.int32),
            pltpu.VMEM((8, N), jnp.int32),
            pltpu.VMEM((4 * _CK, D), jnp.float32),
            pltpu.VMEM((8 * _CK, D), jnp.float32),
            pltpu.VMEM((4 * _RW,), jnp.float32),
            pltpu.VMEM((4 * _RW,), jnp.float32),
            pltpu.VMEM((4 * _RW,), jnp.int32),
            pltpu.VMEM((_BPW,), jnp.int32),
            pltpu.SemaphoreType.DMA((2,)),
            pltpu.SemaphoreType.DMA((2,)),
        ],
    )(_sc_body)
    return f(past_lengths, past_ids, ratings, past_embeddings, posm, t7)


def kernel(past_lengths, past_ids, past_embeddings, ratings, pos_emb,
           iasig_emb, rating_emb):
    posm = (pos_emb + iasig_emb[jnp.arange(2 * N) % 2]).reshape(N, 2 * D)
    t7 = jnp.concatenate([rating_emb * _SCALE,
                          jnp.zeros((1, D), jnp.float32)], axis=0)
    l2, ue, me, mo = _run_sc(past_lengths, past_ids, ratings,
                             past_embeddings, posm, t7)
    m = jnp.stack([me.reshape(B, N), mo.reshape(B, N)], axis=-1)
    return (l2, ue, m.reshape(B, 2 * N, 1))


# speedup vs baseline: 2.7524x; 2.7524x over previous
import functools

import jax
import jax.numpy as jnp
from jax import lax
from jax.experimental import pallas as pl
from jax.experimental.pallas import tpu as pltpu
from jax.experimental.pallas import tpu_sc as plsc

B, N, D = 1024, 200, 128
_SCALE = float(D) ** 0.5

_NW = 32
_BPW = B // _NW
_CK = 40
_NCK = N // _CK
_T = _BPW * _NCK
_RW = 224


def _sc_body(len_hbm, ids_hbm, r_hbm, pe_hbm, posm_hbm, t7_hbm,
             l2_hbm, ue_hbm, me_hbm, mo_hbm,
             posm_v, t7_v, idsb_v, rb_v, pe_v, out_v, me_v, mo_v, rch_v,
             len_v, in_sem, out_sem):
    c = lax.axis_index("c")
    s = lax.axis_index("s")
    w = s * 2 + c
    b0 = pl.multiple_of(w * _BPW, 8)

    pltpu.sync_copy(posm_hbm, posm_v)
    pltpu.sync_copy(t7_hbm, t7_v)
    pltpu.sync_copy(ids_hbm.at[pl.ds(b0, _BPW)], idsb_v)
    pltpu.sync_copy(r_hbm.at[pl.ds(b0, _BPW)], rb_v)

    pltpu.sync_copy(len_hbm.at[pl.ds(b0, _BPW)], len_v)
    for g in range(_BPW // 16):
        len_v[pl.ds(g * 16, 16)] = len_v[pl.ds(g * 16, 16)] * 2
    pltpu.sync_copy(len_v, l2_hbm.at[pl.ds(b0, _BPW)])

    def in_copies(t, slot):
        b = b0 + t // _NCK
        i0 = pl.multiple_of((t % _NCK) * _CK, 8)
        return (
            pltpu.make_async_copy(pe_hbm.at[b, pl.ds(i0, _CK)],
                                  pe_v.at[pl.ds(pl.multiple_of(
                                      slot * _CK, 8), _CK)],
                                  in_sem.at[slot]),
        )

    def out_copies(t, slot):
        bl = t // _NCK
        b = b0 + bl
        i0 = pl.multiple_of((t % _NCK) * _CK, 8)
        fo = pl.multiple_of(b * N + i0, 8)
        mb = pl.multiple_of(lax.rem(bl, 2) * _RW + i0, 8)
        return (
            pltpu.make_async_copy(out_v.at[pl.ds(pl.multiple_of(
                                      slot * 2 * _CK, 8), 2 * _CK)],
                                  ue_hbm.at[b, pl.ds(2 * i0, 2 * _CK)],
                                  out_sem.at[slot]),
            pltpu.make_async_copy(me_v.at[pl.ds(mb, _CK)],
                                  me_hbm.at[pl.ds(fo, _CK)],
                                  out_sem.at[slot]),
            pltpu.make_async_copy(mo_v.at[pl.ds(mb, _CK)],
                                  mo_hbm.at[pl.ds(fo, _CK)],
                                  out_sem.at[slot]),
        )

    def compute(t, slot):
        bl = t // _NCK
        i0 = (t % _NCK) * _CK
        sb = lax.rem(bl, 2) * _RW
        rb = slot * _CK
        rb2 = slot * 2 * _CK

        @pl.when(i0 == 0)
        def _stage_row():
            for goff in list(range(0, N - 16, 16)) + [N - 16]:
                ids16 = idsb_v[bl, pl.ds(goff, 16)]
                r16 = rb_v[bl, pl.ds(goff, 16)]
                me_v[pl.ds(sb + goff, 16)] = jnp.where(
                    ids16 != 0, 1.0, 0.0).astype(jnp.float32)
                mo_v[pl.ds(sb + goff, 16)] = jnp.where(
                    (r16 != 0) & (r16 != 6), 1.0, 0.0).astype(jnp.float32)
                rch_v[pl.ds(sb + goff, 16)] = r16

        @plsc.parallel_loop(0, _CK, 1, unroll=4)
        def pair(j):
            me_s = lax.broadcast(me_v[pl.ds(sb + i0 + j, 16)][0], (16,))
            mo_s = lax.broadcast(mo_v[pl.ds(sb + i0 + j, 16)][0], (16,))
            r = rch_v[pl.ds(sb + i0 + j, 16)][0]
            for k in range(8):
                ev = pe_v[rb + j, pl.ds(k * 16, 16)] * _SCALE
                ev = (ev + posm_v[i0 + j, pl.ds(k * 16, 16)]) * me_s
                out_v[rb2 + 2 * j, pl.ds(k * 16, 16)] = ev
                ov = t7_v[r, pl.ds(k * 16, 16)]
                ov = (ov + posm_v[i0 + j, pl.ds(D + k * 16, 16)]) * mo_s
                out_v[rb2 + 2 * j + 1, pl.ds(k * 16, 16)] = ov

    for dma in in_copies(0, 0):
        dma.start()

    def step(t, _):
        slot = lax.rem(t, 2)

        @pl.when(t + 1 < _T)
        def _prefetch():
            for dma in in_copies(t + 1, 1 - slot):
                dma.start()

        for dma in in_copies(t, slot):
            dma.wait()

        @pl.when(t >= 2)
        def _drain_out():
            for dma in out_copies(t - 2, slot):
                dma.wait()

        compute(t, slot)
        for dma in out_copies(t, slot):
            dma.start()
        return _

    lax.fori_loop(0, _T, step, 0)
    for dma in out_copies(_T - 2, 0):
        dma.wait()
    for dma in out_copies(_T - 1, 1):
        dma.wait()


@jax.jit
def _run_sc(past_lengths, past_ids, ratings, past_embeddings, posm, t7):
    mesh = plsc.VectorSubcoreMesh(core_axis_name="c", subcore_axis_name="s")
    f = functools.partial(
        pl.kernel,
        mesh=mesh,
        out_type=[
            jax.ShapeDtypeStruct((B,), jnp.int32),
            jax.ShapeDtypeStruct((B, 2 * N, D), jnp.float32),
            jax.ShapeDtypeStruct((B * N,), jnp.float32),
            jax.ShapeDtypeStruct((B * N,), jnp.float32),
        ],
        scratch_types=[
            pltpu.VMEM((N, 2 * D), jnp.float32),
            pltpu.VMEM((8, D), jnp.float32),
            pltpu.VMEM((_BPW, N), jnp.int32),
            pltpu.VMEM((_BPW, N), jnp.int32),
            pltpu.VMEM((2 * _CK, D), jnp.float32),
            pltpu.VMEM((4 * _CK, D), jnp.float32),
            pltpu.VMEM((2 * _RW,), jnp.float32),
            pltpu.VMEM((2 * _RW,), jnp.float32),
            pltpu.VMEM((2 * _RW,), jnp.int32),
            pltpu.VMEM((_BPW,), jnp.int32),
            pltpu.SemaphoreType.DMA((2,)),
            pltpu.SemaphoreType.DMA((2,)),
        ],
    )(_sc_body)
    return f(past_lengths, past_ids, ratings, past_embeddings, posm, t7)


def kernel(past_lengths, past_ids, past_embeddings, ratings, pos_emb,
           iasig_emb, rating_emb):
    posm = (pos_emb + iasig_emb[jnp.arange(2 * N) % 2]).reshape(N, 2 * D)
    t7 = jnp.concatenate([rating_emb * _SCALE,
                          jnp.zeros((1, D), jnp.float32)], axis=0)
    l2, ue, me, mo = _run_sc(past_lengths, past_ids, ratings,
                             past_embeddings, posm, t7)
    m = jnp.stack([me.reshape(B, N), mo.reshape(B, N)], axis=-1)
    return (l2, ue, m.reshape(B, 2 * N, 1))
